# baseline (speedup 1.0000x reference)
import functools

import jax
import jax.numpy as jnp
from jax import lax
from jax.experimental import pallas as pl
from jax.experimental.pallas import tpu as pltpu
from jax.experimental.pallas import tpu_sc as plsc

BATCH = 16384
D = 16
CHUNK = 128


def _constrain(W):
    c = jnp.linalg.norm(W, ord=2, axis=1, keepdims=True)
    c = jnp.where(c < 1.0, 1.0, c)
    return W / c


def _rsqrt_nr(x):
    ib = lax.bitcast_convert_type(x, jnp.int32)
    m = jnp.int32(0x5F3759DF) - lax.shift_right_arithmetic(ib, 1)
    y = lax.bitcast_convert_type(m, jnp.float32)
    y = y * (1.5 - 0.5 * x * y * y)
    y = y * (1.5 - 0.5 * x * y * y)
    return y


def _make_sc_kernel(n_workers, b_per_w):
    n_chunks = b_per_w // CHUNK
    mesh = plsc.VectorSubcoreMesh(core_axis_name="c", subcore_axis_name="s")

    @functools.partial(
        pl.kernel,
        mesh=mesh,
        compiler_params=pltpu.CompilerParams(
            needs_layout_passes=False, use_tc_tiling_on_sc=False),
        out_type=(
            jax.ShapeDtypeStruct((BATCH,), jnp.float32),
            jax.ShapeDtypeStruct((n_workers, D), jnp.float32),
            jax.ShapeDtypeStruct((n_workers, D), jnp.float32),
        ),
        scratch_types=[
            pltpu.VMEM((14, D), jnp.float32),
            pltpu.VMEM((n_chunks, CHUNK), jnp.int32),
            pltpu.VMEM((n_chunks, CHUNK), jnp.int32),
            pltpu.VMEM((b_per_w, D), jnp.float32),
            pltpu.VMEM((b_per_w, D), jnp.float32),
            pltpu.VMEM((b_per_w,), jnp.float32),
            pltpu.VMEM((D,), jnp.float32),
            pltpu.VMEM((D,), jnp.float32),
            pltpu.SemaphoreType.DMA,
        ],
    )
    def k(users_hbm, items_hbm, u_tab, i_tab, params_hbm,
          out_hbm, pu_hbm, pi_hbm,
          params_v, idx_u, idx_i, rows_u, rows_i, out_v, accu_v, acci_v, sem):
        nc = lax.axis_index("c")
        ns = lax.axis_index("s")
        wid = ns * 2 + nc
        base = wid * b_per_w

        for j in range(n_chunks):
            pltpu.sync_copy(users_hbm.at[pl.ds(base + j * CHUNK, CHUNK)], idx_u.at[j])
            pltpu.sync_copy(items_hbm.at[pl.ds(base + j * CHUNK, CHUNK)], idx_i.at[j])
        pltpu.sync_copy(params_hbm, params_v)

        copies = []
        for j in range(n_chunks):
            copies.append(pltpu.async_copy(
                u_tab.at[idx_u.at[j]], rows_u.at[pl.ds(j * CHUNK, CHUNK)], sem))
            copies.append(pltpu.async_copy(
                i_tab.at[idx_i.at[j]], rows_i.at[pl.ds(j * CHUNK, CHUNK)], sem))
        for c in copies:
            c.wait()

        wu_v, wi_v, wm_v, wd_v = (params_v[r, :] for r in range(4))
        wus = [wu_v[c] for c in range(D)]
        wis = [wi_v[c] for c in range(D)]
        wms = [wm_v[c] for c in range(D)]
        wds = [wd_v[c] for c in range(D)]
        u0 = params_v[4, :]
        u1 = params_v[5, :]
        u2 = params_v[6, :]
        cp = params_v[7, :]
        sp = params_v[8, :]
        q0 = params_v[9, :]
        q1 = params_v[10, :]
        q2 = params_v[11, :]
        cq = params_v[12, :]
        sq = params_v[13, :]
        lane = lax.iota(jnp.int32, 16)

        def trans(e, t0, t1, t2, ca, sa):
            ab = jnp.abs(e)
            x = ab + 1e-7
            s = x * _rsqrt_nr(x)
            sqr = e * e
            unary = t0 * s + t1 * ab + t2 * sqr
            assist = ca + sa * jnp.sign(e)
            return unary * assist, sqr

        def group_body(g, carry):
            au, ai = carry
            ridx = g * 16 + lane
            res = jnp.zeros((16,), jnp.float32)
            for c in range(D):
                cidx = jnp.full((16,), c, jnp.int32)
                cu = plsc.load_gather(rows_u, [ridx, cidx])
                ci = plsc.load_gather(rows_i, [ridx, cidx])
                tu, squ = trans(cu, u0, u1, u2, cp, sp)
                ti, sqi = trans(ci, q0, q1, q2, cq, sq)
                au = au + squ
                ai = ai + sqi
                res = (res + tu * wus[c] + ti * wis[c]
                       + (tu * ti) * wms[c] + jnp.abs(tu - ti) * wds[c])
            out_v[pl.ds(g * 16, 16)] = res
            return au, ai

        zero = jnp.zeros((16,), jnp.float32)
        au, ai = lax.fori_loop(0, b_per_w // 16, group_body, (zero, zero))
        accu_v[...] = au
        acci_v[...] = ai

        pltpu.sync_copy(out_v, out_hbm.at[pl.ds(base, b_per_w)])
        pltpu.sync_copy(accu_v, pu_hbm.at[wid])
        pltpu.sync_copy(acci_v, pi_hbm.at[wid])

    return k


def kernel(users, items, U, I, a_unary_p, a_unary_q, a_assist_p, a_assist_q,
           a_binary, W0, W1, W2, W3, W4):
    W0, W1, W2, W3, W4 = map(_constrain, (W0, W1, W2, W3, W4))
    a = a_binary
    half = 0.5 * (a[2] * W2[0] + a[3] * W3[0])
    wu = a[0] * W0[0] + half + a[4] * W4[0, :D]
    wi = a[0] * W0[0] + half + a[4] * W4[0, D:]
    wm = a[1] * W1[0]
    wd = 0.5 * (a[2] * W2[0] - a[3] * W3[0])
    sp = jax.nn.softmax(a_assist_p)
    sq = jax.nn.softmax(a_assist_q)

    def splat(s):
        return jnp.full((D,), s, jnp.float32)

    params = jnp.stack([
        wu, wi, wm, wd,
        splat(a_unary_p[0]), splat(a_unary_p[1]), splat(a_unary_p[2]),
        splat(sp[0] - sp[1]), splat(sp[2]),
        splat(a_unary_q[0]), splat(a_unary_q[1]), splat(a_unary_q[2]),
        splat(sq[0] - sq[1]), splat(sq[2]),
    ])

    info = plsc.get_sparse_core_info()
    n_workers = info.num_cores * info.num_subcores
    b_per_w = BATCH // n_workers

    k = _make_sc_kernel(n_workers, b_per_w)
    out, pu, pi = k(users.astype(jnp.int32), items.astype(jnp.int32),
                    U, I, params)

    inferences = out.reshape(BATCH, 1)
    regs = 0.01 * (jnp.sqrt(jnp.sum(pu)) + jnp.sqrt(jnp.sum(pi)))
    return inferences, regs
